# baseline (speedup 1.0000x reference)
import jax
import jax.numpy as jnp
from jax import lax
from jax.experimental import pallas as pl
from jax.experimental.pallas import tpu as pltpu
from jax.experimental.pallas import tpu_sc as plsc

B = 4096
K = 32

_info = plsc.get_sparse_core_info()
_NC = _info.num_cores
_NS = _info.num_subcores
_NW = _NC * _NS
_BPW = B // _NW


def _item_gather_body(item_hbm, itab_hbm, iout_hbm, iidx_v, irows_v, sem):
  wid = lax.axis_index("s") * _NC + lax.axis_index("c")
  base = wid * _BPW
  pltpu.sync_copy(item_hbm.at[pl.ds(base, _BPW)], iidx_v)
  pltpu.async_copy(itab_hbm.at[iidx_v], irows_v, sem).wait()
  pltpu.sync_copy(irows_v, iout_hbm.at[pl.ds(base, _BPW)])


_item_gather = pl.kernel(
    _item_gather_body,
    out_type=jax.ShapeDtypeStruct((B, K), jnp.float32),
    mesh=plsc.VectorSubcoreMesh(core_axis_name="c", subcore_axis_name="s"),
    scratch_types=[
        pltpu.VMEM((_BPW,), jnp.int32),
        pltpu.VMEM((_BPW, K), jnp.float32),
        pltpu.SemaphoreType.DMA,
    ],
    compiler_params=pltpu.CompilerParams(use_tc_tiling_on_sc=False),
)


_BM = 256
_NT = B // _BM


def _residual_body(uidx_ref, rating_ref, v_ref, utab_ref, out_ref,
                   u_scr, sem0, sem1):
  i = pl.program_id(0)
  buf = lax.rem(i, 2)

  def enqueue(step, bslot, sem):
    base = step * _BM

    def body(j, carry):
      row = uidx_ref[base + j]
      pltpu.make_async_copy(
          utab_ref.at[pl.ds(row, 1), :],
          u_scr.at[bslot, pl.ds(j, 1), :],
          sem).start()
      return carry

    lax.fori_loop(0, _BM, body, 0)

  @pl.when(i == 0)
  def _():
    enqueue(0, 0, sem0)
    enqueue(1, 1, sem1)

  @pl.when((i > 0) & (i + 1 < _NT))
  def _():
    nbuf = lax.rem(i + 1, 2)

    @pl.when(nbuf == 0)
    def _():
      enqueue(i + 1, 0, sem0)

    @pl.when(nbuf == 1)
    def _():
      enqueue(i + 1, 1, sem1)

  @pl.when(buf == 0)
  def _():
    pltpu.make_async_copy(
        utab_ref.at[pl.ds(0, _BM), :], u_scr.at[0], sem0).wait()

  @pl.when(buf == 1)
  def _():
    pltpu.make_async_copy(
        utab_ref.at[pl.ds(0, _BM), :], u_scr.at[1], sem1).wait()

  preds = lax.dot_general(
      u_scr[buf], v_ref[...],
      dimension_numbers=(((1,), (1,)), ((), ())),
      preferred_element_type=jnp.float32)
  out_ref[...] = rating_ref[...] - preds


def _residual(user, rating, i_emb, user_table):
  return pl.pallas_call(
      _residual_body,
      grid=(_NT,),
      in_specs=[
          pl.BlockSpec(memory_space=pltpu.SMEM),
          pl.BlockSpec((_BM, B), lambda i: (i, 0)),
          pl.BlockSpec((B, K), lambda i: (0, 0)),
          pl.BlockSpec(memory_space=pl.ANY),
      ],
      out_specs=pl.BlockSpec((_BM, B), lambda i: (i, 0)),
      out_shape=jax.ShapeDtypeStruct((B, B), jnp.float32),
      scratch_shapes=[
          pltpu.VMEM((2, _BM, K), jnp.float32),
          pltpu.SemaphoreType.DMA,
          pltpu.SemaphoreType.DMA,
      ],
      compiler_params=pltpu.CompilerParams(
          dimension_semantics=("arbitrary",)),
  )(user, rating, i_emb, user_table)


@jax.jit
def kernel(user, item, rating, user_table, item_table):
  i_emb = _item_gather(item.astype(jnp.int32), item_table)
  return _residual(user.astype(jnp.int32), rating, i_emb, user_table)


# speedup vs baseline: 3.3485x; 3.3485x over previous
import jax
import jax.numpy as jnp
from jax import lax
from jax.experimental import pallas as pl
from jax.experimental.pallas import tpu as pltpu
from jax.experimental.pallas import tpu_sc as plsc

B = 4096
K = 32

_info = plsc.get_sparse_core_info()
_NC = _info.num_cores
_NS = _info.num_subcores
_NW = _NC * _NS
_BPW = B // _NW


def _item_gather_body(item_hbm, itab_hbm, iout_hbm, iidx_v, irows_v, sem):
  wid = lax.axis_index("s") * _NC + lax.axis_index("c")
  base = wid * _BPW
  pltpu.sync_copy(item_hbm.at[pl.ds(base, _BPW)], iidx_v)
  pltpu.async_copy(itab_hbm.at[iidx_v], irows_v, sem).wait()
  pltpu.sync_copy(irows_v, iout_hbm.at[pl.ds(base, _BPW)])


_item_gather = pl.kernel(
    _item_gather_body,
    out_type=jax.ShapeDtypeStruct((B, K), jnp.float32),
    mesh=plsc.VectorSubcoreMesh(core_axis_name="c", subcore_axis_name="s"),
    scratch_types=[
        pltpu.VMEM((_BPW,), jnp.int32),
        pltpu.VMEM((_BPW, K), jnp.float32),
        pltpu.SemaphoreType.DMA,
    ],
    compiler_params=pltpu.CompilerParams(use_tc_tiling_on_sc=False),
)


_BM = 256
_NT = B // _BM


def _residual_body(rating_ref, u_ref, v_ref, out_ref):
  preds = lax.dot_general(
      u_ref[...], v_ref[...],
      dimension_numbers=(((1,), (1,)), ((), ())),
      preferred_element_type=jnp.float32)
  out_ref[...] = rating_ref[...] - preds


def _residual(rating, u_emb, i_emb):
  return pl.pallas_call(
      _residual_body,
      grid=(_NT,),
      in_specs=[
          pl.BlockSpec((_BM, B), lambda i: (i, 0)),
          pl.BlockSpec((_BM, K), lambda i: (i, 0)),
          pl.BlockSpec((B, K), lambda i: (0, 0)),
      ],
      out_specs=pl.BlockSpec((_BM, B), lambda i: (i, 0)),
      out_shape=jax.ShapeDtypeStruct((B, B), jnp.float32),
  )(rating, u_emb, i_emb)


@jax.jit
def kernel(user, item, rating, user_table, item_table):
  i_emb = _item_gather(item.astype(jnp.int32), item_table)
  u_emb = jnp.take(user_table, user, axis=0)
  return _residual(rating, u_emb, i_emb)
